# baseline (speedup 1.0000x reference)
import functools

import jax
import jax.numpy as jnp
import numpy as np
from jax import lax
from jax.experimental import pallas as pl
from jax.experimental.pallas import tpu as pltpu
from jax.experimental.pallas import tpu_sc as plsc

_N_LEVELS = 16
_T = 1 << 19
_BASE_RES = 16
_P1 = np.int32(np.uint32(2654435761).astype(np.int32))
_MASK = np.int32(_T - 1)
_N_PIX = 262144
_N_CHUNK = 4
_CHUNK = _N_PIX // _N_CHUNK
_NW = 32
_PW = _CHUNK // _NW
_BLK = 256
_NB = _PW // _BLK
_RES = [int(np.floor(_BASE_RES * (2.0 ** l))) for l in range(_N_LEVELS)]
_SMALL = [0, 1, 2, 3]
_STREAM = list(range(4, 16))
_NSMALL = [(_RES[l] + 1) ** 2 for l in _SMALL]
_SOFS = [0]
for _n in _NSMALL[:-1]:
    _SOFS.append(_SOFS[-1] + _n)
_BLOB = sum(_NSMALL) * 2
_CORNERS = [(0, 0), (0, 1), (1, 0), (1, 1)]
_MLP_BLK = 8192


def _coords(xs_v, ys_v, g, res):
    x = xs_v[pl.ds(g * 16, 16)]
    y = ys_v[pl.ds(g * 16, 16)]
    res_f = jnp.float32(res)
    posx = x * res_f
    posy = y * res_f
    ix = posx.astype(jnp.int32)
    iy = posy.astype(jnp.int32)
    wx1 = posx - ix.astype(jnp.float32)
    wy1 = posy - iy.astype(jnp.float32)
    return ix, iy, wx1, wy1, 1.0 - wx1, 1.0 - wy1


def _encoder(xs, ys, ht0, ht1, blob):
    mesh = plsc.VectorSubcoreMesh(core_axis_name="c", subcore_axis_name="s")

    @functools.partial(
        pl.kernel,
        out_type=jax.ShapeDtypeStruct((2 * _N_LEVELS, _CHUNK), jnp.float32),
        mesh=mesh,
        compiler_params=pltpu.CompilerParams(
            needs_layout_passes=False, use_tc_tiling_on_sc=False),
        scratch_types=[
            pltpu.VMEM((_BLK,), jnp.float32),
            pltpu.VMEM((_BLK,), jnp.float32),
            pltpu.VMEM((_BLOB // 8, 8), jnp.float32),
            pltpu.VMEM((48 * _BLK,), jnp.int32),
            pltpu.VMEM((48, _BLK), jnp.float32),
            pltpu.VMEM((48 * _BLK,), jnp.float32),
            pltpu.VMEM((48 * _BLK,), jnp.float32),
            pltpu.VMEM((2 * _N_LEVELS, _BLK), jnp.float32),
            pltpu.SemaphoreType.DMA,
        ],
    )
    def enc(xs_hbm, ys_hbm, ht0_hbm, ht1_hbm, blob_hbm, out_hbm,
            xs_v, ys_v, dense_v, idxbuf, wbuf, rows0, rows1, featb, sem):
        wid = lax.axis_index("s") * 2 + lax.axis_index("c")
        base = wid * _PW
        pltpu.sync_copy(blob_hbm, dense_v)
        iota16 = lax.iota(jnp.int32, 16)

        def block_body(b, carry):
            row0 = base + b * _BLK
            pltpu.sync_copy(xs_hbm.at[pl.ds(row0, _BLK)], xs_v)
            pltpu.sync_copy(ys_hbm.at[pl.ds(row0, _BLK)], ys_v)

            descs = []
            for l in _STREAM:
                k0 = 4 * (l - _STREAM[0])
                res = _RES[l]
                dense = (res + 1) ** 2 <= _T
                lofs = jnp.int32(l * _T)

                def grp_a(g, _, l=l, k0=k0, res=res, dense=dense, lofs=lofs):
                    ix, iy, wx1, wy1, wx0, wy0 = _coords(xs_v, ys_v, g, res)
                    if dense:
                        stride = jnp.int32(res + 1)
                        b00 = ix * stride + iy + lofs
                        idxs = [b00, b00 + 1, b00 + stride, b00 + stride + 1]
                    else:
                        hy0 = iy * _P1
                        hy1 = hy0 + _P1
                        ix1 = ix + 1
                        idxs = [
                            (lax.bitwise_and(lax.bitwise_xor(cx, hy), _MASK))
                            + lofs
                            for cx, hy in ((ix, hy0), (ix, hy1),
                                           (ix1, hy0), (ix1, hy1))
                        ]
                    ws = [wx0 * wy0, wx0 * wy1, wx1 * wy0, wx1 * wy1]
                    for ci in range(4):
                        idxbuf[pl.ds((k0 + ci) * _BLK + g * 16, 16)] = (
                            idxs[ci])
                        wbuf[k0 + ci, pl.ds(g * 16, 16)] = ws[ci]
                    return _

                lax.fori_loop(0, _BLK // 16, grp_a, 0)
                lsl = pl.ds(k0 * _BLK, 4 * _BLK)
                descs.append(pltpu.async_copy(
                    ht0_hbm.at[idxbuf.at[lsl]], rows0.at[lsl], sem))
                descs.append(pltpu.async_copy(
                    ht1_hbm.at[idxbuf.at[lsl]], rows1.at[lsl], sem))

            for li, l in enumerate(_SMALL):
                res = _RES[l]
                stride = jnp.int32(res + 1)
                eofs = jnp.int32(_SOFS[li])

                def grp_s(g, _, l=l, res=res, stride=stride, eofs=eofs):
                    ix, iy, wx1, wy1, wx0, wy0 = _coords(xs_v, ys_v, g, res)
                    e00 = ix * stride + iy + eofs
                    es = [e00, e00 + 1, e00 + stride, e00 + stride + 1]
                    ws = [wx0 * wy0, wx0 * wy1, wx1 * wy0, wx1 * wy1]
                    acc0 = jnp.zeros((16,), jnp.float32)
                    acc1 = jnp.zeros((16,), jnp.float32)
                    for ci in range(4):
                        er = lax.shift_right_logical(es[ci], 2)
                        ec = lax.bitwise_and(es[ci], 3) * 2
                        f0 = plsc.load_gather(dense_v, [er, ec])
                        f1 = plsc.load_gather(dense_v, [er, ec + 1])
                        acc0 = acc0 + f0 * ws[ci]
                        acc1 = acc1 + f1 * ws[ci]
                    featb[2 * l, pl.ds(g * 16, 16)] = acc0
                    featb[2 * l + 1, pl.ds(g * 16, 16)] = acc1
                    return _

                lax.fori_loop(0, _BLK // 16, grp_s, 0)

            for d in descs:
                d.wait()

            for l in _STREAM:
                k0 = 4 * (l - _STREAM[0])

                def grp_c(g, _, l=l, k0=k0):
                    r0 = g * 16
                    acc0 = jnp.zeros((16,), jnp.float32)
                    acc1 = jnp.zeros((16,), jnp.float32)
                    for ci in range(4):
                        w = wbuf[k0 + ci, pl.ds(r0, 16)]
                        f0 = rows0[pl.ds((k0 + ci) * _BLK + r0, 16)]
                        f1 = rows1[pl.ds((k0 + ci) * _BLK + r0, 16)]
                        acc0 = acc0 + f0 * w
                        acc1 = acc1 + f1 * w
                    featb[2 * l, pl.ds(r0, 16)] = acc0
                    featb[2 * l + 1, pl.ds(r0, 16)] = acc1
                    return _

                lax.fori_loop(0, _BLK // 16, grp_c, 0)

            pltpu.sync_copy(featb, out_hbm.at[:, pl.ds(row0, _BLK)])
            return carry

        lax.fori_loop(0, _NB, block_body, 0)

    return enc(xs, ys, ht0, ht1, blob)


def _mlp_body(f_ref, w1_ref, w2_ref, w3_ref, o_ref):
    h = lax.dot_general(f_ref[...], w1_ref[...], (((0,), (0,)), ((), ())),
                        preferred_element_type=jnp.float32)
    h = jnp.maximum(h, 0.0)
    h = lax.dot_general(h, w2_ref[...], (((1,), (0,)), ((), ())),
                        preferred_element_type=jnp.float32)
    h = jnp.maximum(h, 0.0)
    o_ref[...] = lax.dot_general(h, w3_ref[...], (((1,), (0,)), ((), ())),
                                 preferred_element_type=jnp.float32)


def _mlp(feats, W1, W2, W3p):
    d_in = 2 * _N_LEVELS
    return pl.pallas_call(
        _mlp_body,
        grid=(_CHUNK // _MLP_BLK,),
        in_specs=[
            pl.BlockSpec((d_in, _MLP_BLK), lambda i: (0, i)),
            pl.BlockSpec((d_in, 64), lambda i: (0, 0)),
            pl.BlockSpec((64, 64), lambda i: (0, 0)),
            pl.BlockSpec((64, 8), lambda i: (0, 0)),
        ],
        out_specs=pl.BlockSpec((_MLP_BLK, 8), lambda i: (i, 0)),
        out_shape=jax.ShapeDtypeStruct((_CHUNK, 8), jnp.float32),
    )(feats, W1, W2, W3p)


def kernel(v_pixel_pos, hash_tables, W1, W2, W3):
    xs = v_pixel_pos[:, 0]
    ys = v_pixel_pos[:, 1]
    ht0 = hash_tables[:, :, 0].reshape(_N_LEVELS * _T)
    ht1 = hash_tables[:, :, 1].reshape(_N_LEVELS * _T)
    blob = jnp.concatenate(
        [hash_tables[l, :_NSMALL[li]] for li, l in enumerate(_SMALL)]
    ).reshape(_BLOB // 8, 8)
    W3p = jnp.pad(W3, ((0, 0), (0, 8 - W3.shape[1])))
    feats_list = []
    for c in range(_N_CHUNK):
        sl = slice(c * _CHUNK, (c + 1) * _CHUNK)
        feats_list.append(_encoder(xs[sl], ys[sl], ht0, ht1, blob))
    outs = [_mlp(f, W1, W2, W3p) for f in feats_list]
    return jnp.concatenate(outs, axis=0)[:, :3]


# speedup vs baseline: 1.0115x; 1.0115x over previous
import functools

import jax
import jax.numpy as jnp
import numpy as np
from jax import lax
from jax.experimental import pallas as pl
from jax.experimental.pallas import tpu as pltpu
from jax.experimental.pallas import tpu_sc as plsc

_N_LEVELS = 16
_T = 1 << 19
_BASE_RES = 16
_P1 = np.int32(np.uint32(2654435761).astype(np.int32))
_MASK = np.int32(_T - 1)
_N_PIX = 262144
_N_CHUNK = 8
_CHUNK = _N_PIX // _N_CHUNK
_NW = 32
_PW = _CHUNK // _NW
_BLK = 256
_NB = _PW // _BLK
_RES = [int(np.floor(_BASE_RES * (2.0 ** l))) for l in range(_N_LEVELS)]
_SMALL = [0, 1, 2, 3]
_STREAM = list(range(4, 16))
_NSMALL = [(_RES[l] + 1) ** 2 for l in _SMALL]
_SOFS = [0]
for _n in _NSMALL[:-1]:
    _SOFS.append(_SOFS[-1] + _n)
_BLOB = sum(_NSMALL) * 2
_CORNERS = [(0, 0), (0, 1), (1, 0), (1, 1)]
_MLP_BLK = 8192


def _coords(xs_v, ys_v, g, res):
    x = xs_v[pl.ds(g * 16, 16)]
    y = ys_v[pl.ds(g * 16, 16)]
    res_f = jnp.float32(res)
    posx = x * res_f
    posy = y * res_f
    ix = posx.astype(jnp.int32)
    iy = posy.astype(jnp.int32)
    wx1 = posx - ix.astype(jnp.float32)
    wy1 = posy - iy.astype(jnp.float32)
    return ix, iy, wx1, wy1, 1.0 - wx1, 1.0 - wy1


def _encoder(xs, ys, ht0, ht1, blob):
    mesh = plsc.VectorSubcoreMesh(core_axis_name="c", subcore_axis_name="s")

    @functools.partial(
        pl.kernel,
        out_type=jax.ShapeDtypeStruct((2 * _N_LEVELS, _CHUNK), jnp.float32),
        mesh=mesh,
        compiler_params=pltpu.CompilerParams(
            needs_layout_passes=False, use_tc_tiling_on_sc=False),
        scratch_types=[
            pltpu.VMEM((_BLK,), jnp.float32),
            pltpu.VMEM((_BLK,), jnp.float32),
            pltpu.VMEM((_BLOB // 8, 8), jnp.float32),
            pltpu.VMEM((48 * _BLK,), jnp.int32),
            pltpu.VMEM((48, _BLK), jnp.float32),
            pltpu.VMEM((48 * _BLK,), jnp.float32),
            pltpu.VMEM((48 * _BLK,), jnp.float32),
            pltpu.VMEM((2 * _N_LEVELS, _BLK), jnp.float32),
            pltpu.SemaphoreType.DMA,
        ],
    )
    def enc(xs_hbm, ys_hbm, ht0_hbm, ht1_hbm, blob_hbm, out_hbm,
            xs_v, ys_v, dense_v, idxbuf, wbuf, rows0, rows1, featb, sem):
        wid = lax.axis_index("s") * 2 + lax.axis_index("c")
        base = wid * _PW
        pltpu.sync_copy(blob_hbm, dense_v)
        iota16 = lax.iota(jnp.int32, 16)

        def block_body(b, carry):
            row0 = base + b * _BLK
            pltpu.sync_copy(xs_hbm.at[pl.ds(row0, _BLK)], xs_v)
            pltpu.sync_copy(ys_hbm.at[pl.ds(row0, _BLK)], ys_v)

            descs = []
            for l in _STREAM:
                k0 = 4 * (l - _STREAM[0])
                res = _RES[l]
                dense = (res + 1) ** 2 <= _T
                lofs = jnp.int32(l * _T)

                def grp_a(g, _, l=l, k0=k0, res=res, dense=dense, lofs=lofs):
                    ix, iy, wx1, wy1, wx0, wy0 = _coords(xs_v, ys_v, g, res)
                    if dense:
                        stride = jnp.int32(res + 1)
                        b00 = ix * stride + iy + lofs
                        idxs = [b00, b00 + 1, b00 + stride, b00 + stride + 1]
                    else:
                        hy0 = iy * _P1
                        hy1 = hy0 + _P1
                        ix1 = ix + 1
                        idxs = [
                            (lax.bitwise_and(lax.bitwise_xor(cx, hy), _MASK))
                            + lofs
                            for cx, hy in ((ix, hy0), (ix, hy1),
                                           (ix1, hy0), (ix1, hy1))
                        ]
                    ws = [wx0 * wy0, wx0 * wy1, wx1 * wy0, wx1 * wy1]
                    for ci in range(4):
                        idxbuf[pl.ds((k0 + ci) * _BLK + g * 16, 16)] = (
                            idxs[ci])
                        wbuf[k0 + ci, pl.ds(g * 16, 16)] = ws[ci]
                    return _

                lax.fori_loop(0, _BLK // 16, grp_a, 0)
                lsl = pl.ds(k0 * _BLK, 4 * _BLK)
                descs.append(pltpu.async_copy(
                    ht0_hbm.at[idxbuf.at[lsl]], rows0.at[lsl], sem))
                descs.append(pltpu.async_copy(
                    ht1_hbm.at[idxbuf.at[lsl]], rows1.at[lsl], sem))

            for li, l in enumerate(_SMALL):
                res = _RES[l]
                stride = jnp.int32(res + 1)
                eofs = jnp.int32(_SOFS[li])

                def grp_s(g, _, l=l, res=res, stride=stride, eofs=eofs):
                    ix, iy, wx1, wy1, wx0, wy0 = _coords(xs_v, ys_v, g, res)
                    e00 = ix * stride + iy + eofs
                    es = [e00, e00 + 1, e00 + stride, e00 + stride + 1]
                    ws = [wx0 * wy0, wx0 * wy1, wx1 * wy0, wx1 * wy1]
                    acc0 = jnp.zeros((16,), jnp.float32)
                    acc1 = jnp.zeros((16,), jnp.float32)
                    for ci in range(4):
                        er = lax.shift_right_logical(es[ci], 2)
                        ec = lax.bitwise_and(es[ci], 3) * 2
                        f0 = plsc.load_gather(dense_v, [er, ec])
                        f1 = plsc.load_gather(dense_v, [er, ec + 1])
                        acc0 = acc0 + f0 * ws[ci]
                        acc1 = acc1 + f1 * ws[ci]
                    featb[2 * l, pl.ds(g * 16, 16)] = acc0
                    featb[2 * l + 1, pl.ds(g * 16, 16)] = acc1
                    return _

                lax.fori_loop(0, _BLK // 16, grp_s, 0)

            for d in descs:
                d.wait()

            for l in _STREAM:
                k0 = 4 * (l - _STREAM[0])

                def grp_c(g, _, l=l, k0=k0):
                    r0 = g * 16
                    acc0 = jnp.zeros((16,), jnp.float32)
                    acc1 = jnp.zeros((16,), jnp.float32)
                    for ci in range(4):
                        w = wbuf[k0 + ci, pl.ds(r0, 16)]
                        f0 = rows0[pl.ds((k0 + ci) * _BLK + r0, 16)]
                        f1 = rows1[pl.ds((k0 + ci) * _BLK + r0, 16)]
                        acc0 = acc0 + f0 * w
                        acc1 = acc1 + f1 * w
                    featb[2 * l, pl.ds(r0, 16)] = acc0
                    featb[2 * l + 1, pl.ds(r0, 16)] = acc1
                    return _

                lax.fori_loop(0, _BLK // 16, grp_c, 0)

            pltpu.sync_copy(featb, out_hbm.at[:, pl.ds(row0, _BLK)])
            return carry

        lax.fori_loop(0, _NB, block_body, 0)

    return enc(xs, ys, ht0, ht1, blob)


def _mlp_body(f_ref, w1_ref, w2_ref, w3_ref, o_ref):
    h = lax.dot_general(f_ref[...], w1_ref[...], (((0,), (0,)), ((), ())),
                        preferred_element_type=jnp.float32)
    h = jnp.maximum(h, 0.0)
    h = lax.dot_general(h, w2_ref[...], (((1,), (0,)), ((), ())),
                        preferred_element_type=jnp.float32)
    h = jnp.maximum(h, 0.0)
    o_ref[...] = lax.dot_general(h, w3_ref[...], (((1,), (0,)), ((), ())),
                                 preferred_element_type=jnp.float32)


def _mlp(feats, W1, W2, W3p):
    d_in = 2 * _N_LEVELS
    return pl.pallas_call(
        _mlp_body,
        grid=(_CHUNK // _MLP_BLK,),
        in_specs=[
            pl.BlockSpec((d_in, _MLP_BLK), lambda i: (0, i)),
            pl.BlockSpec((d_in, 64), lambda i: (0, 0)),
            pl.BlockSpec((64, 64), lambda i: (0, 0)),
            pl.BlockSpec((64, 8), lambda i: (0, 0)),
        ],
        out_specs=pl.BlockSpec((_MLP_BLK, 8), lambda i: (i, 0)),
        out_shape=jax.ShapeDtypeStruct((_CHUNK, 8), jnp.float32),
    )(feats, W1, W2, W3p)


def kernel(v_pixel_pos, hash_tables, W1, W2, W3):
    xs = v_pixel_pos[:, 0]
    ys = v_pixel_pos[:, 1]
    ht0 = hash_tables[:, :, 0].reshape(_N_LEVELS * _T)
    ht1 = hash_tables[:, :, 1].reshape(_N_LEVELS * _T)
    blob = jnp.concatenate(
        [hash_tables[l, :_NSMALL[li]] for li, l in enumerate(_SMALL)]
    ).reshape(_BLOB // 8, 8)
    W3p = jnp.pad(W3, ((0, 0), (0, 8 - W3.shape[1])))
    feats_list = []
    for c in range(_N_CHUNK):
        sl = slice(c * _CHUNK, (c + 1) * _CHUNK)
        feats_list.append(_encoder(xs[sl], ys[sl], ht0, ht1, blob))
    outs = [_mlp(f, W1, W2, W3p) for f in feats_list]
    return jnp.concatenate(outs, axis=0)[:, :3]
